# baseline (speedup 1.0000x reference)
import functools

import jax
import jax.numpy as jnp
from jax.experimental import pallas as pl
from jax.experimental.pallas import tpu as pltpu

_B, _S, _D, _E = 2, 2048, 1024, 8
_TS = 512


def _router_kernel(h_ref, qw_ref, cw_ref, out_ref):
    s = pl.program_id(0)

    x = h_ref[0]
    q = jax.lax.dot_general(
        x, qw_ref[...], (((1,), (1,)), ((), ())),
        preferred_element_type=jnp.float32)
    qn2 = jnp.sum(q * q, axis=1, keepdims=True)

    lane = jax.lax.broadcasted_iota(jnp.int32, (1, _E), 1)
    acc = jnp.zeros((1, _E), dtype=jnp.float32)
    for e in range(_E):
        k = jax.lax.dot_general(
            x, cw_ref[e], (((1,), (1,)), ((), ())),
            preferred_element_type=jnp.float32)
        num = jnp.sum(q * k, axis=1, keepdims=True)
        kn2 = jnp.sum(k * k, axis=1, keepdims=True)
        denom = jnp.maximum(jnp.sqrt(qn2 * kn2), 1e-8)
        part = jnp.sum(num / denom) * (1.0 / _S)
        acc = acc + jnp.where(lane == e, part, 0.0)

    @pl.when(s == 0)
    def _init():
        out_ref[...] = jnp.zeros_like(out_ref)

    out_ref[...] += acc[None]


def _local_scores(h, q_W, chip_weights):
    local_b = h.shape[0]
    n_s_tiles = _S // _TS
    out = pl.pallas_call(
        _router_kernel,
        grid=(n_s_tiles, local_b),
        in_specs=[
            pl.BlockSpec((1, _TS, _D), lambda s, b: (b, s, 0)),
            pl.BlockSpec((_D, _D), lambda s, b: (0, 0)),
            pl.BlockSpec((_E, _D, _D), lambda s, b: (0, 0, 0)),
        ],
        out_specs=pl.BlockSpec((1, 1, _E), lambda s, b: (b, 0, 0)),
        out_shape=jax.ShapeDtypeStruct((local_b, 1, _E), jnp.float32),
        compiler_params=pltpu.CompilerParams(
            dimension_semantics=("arbitrary", "arbitrary"),
        ),
    )(h, q_W, chip_weights)
    return out.reshape(local_b, _E)


def kernel(h, q_W, chip_weights):
    devs = jax.devices()
    n_dev = 2 if len(devs) >= 2 else 1
    if n_dev == 1:
        return _local_scores(h, q_W, chip_weights)
    mesh = jax.sharding.Mesh(devs[:n_dev], ("x",))
    P = jax.sharding.PartitionSpec
    f = jax.shard_map(
        _local_scores,
        mesh=mesh,
        in_specs=(P("x", None, None), P(None, None), P(None, None, None)),
        out_specs=P("x", None),
        check_vma=False,
    )
    return f(h, q_W, chip_weights)


# speedup vs baseline: 4.2028x; 4.2028x over previous
import jax
import jax.numpy as jnp
from jax.experimental import pallas as pl
from jax.experimental.pallas import tpu as pltpu

_B, _S, _D, _E = 2, 2048, 1024, 8
_TS = 512


def _router_kernel(h_ref, qw_ref, cw_hbm, out_ref, w_vmem, sems):
    b = pl.program_id(0)
    s = pl.program_id(1)
    is_first = (b == 0) & (s == 0)

    @pl.when(is_first)
    def _start_weight_dmas():
        for e in range(_E):
            pltpu.make_async_copy(cw_hbm.at[e], w_vmem.at[e], sems.at[e]).start()

    x = h_ref[0]
    q = jax.lax.dot_general(
        x, qw_ref[...], (((1,), (1,)), ((), ())),
        preferred_element_type=jnp.float32)
    qn2 = jnp.sum(q * q, axis=1, keepdims=True)

    nums = []
    kn2s = []
    for e in range(_E):
        @pl.when(is_first)
        def _wait_w():
            pltpu.make_async_copy(cw_hbm.at[e], w_vmem.at[e], sems.at[e]).wait()

        k = jax.lax.dot_general(
            x, w_vmem[e], (((1,), (1,)), ((), ())),
            preferred_element_type=jnp.float32)
        nums.append(jnp.sum(q * k, axis=1, keepdims=True))
        kn2s.append(jnp.sum(k * k, axis=1, keepdims=True))

    num = jnp.concatenate(nums, axis=1)
    kn2 = jnp.concatenate(kn2s, axis=1)
    denom = jnp.maximum(jnp.sqrt(qn2 * kn2), 1e-8)
    part = jnp.sum(num / denom, axis=0, keepdims=True) * (1.0 / _S)

    @pl.when(s == 0)
    def _init():
        out_ref[...] = jnp.zeros_like(out_ref)

    out_ref[...] += part[None]


def kernel(h, q_W, chip_weights):
    n_s_tiles = _S // _TS
    out = pl.pallas_call(
        _router_kernel,
        grid=(_B, n_s_tiles),
        in_specs=[
            pl.BlockSpec((1, _TS, _D), lambda b, s: (b, s, 0)),
            pl.BlockSpec((_D, _D), lambda b, s: (0, 0)),
            pl.BlockSpec(memory_space=pltpu.MemorySpace.HBM),
        ],
        out_specs=pl.BlockSpec((1, 1, _E), lambda b, s: (b, 0, 0)),
        out_shape=jax.ShapeDtypeStruct((_B, 1, _E), jnp.float32),
        scratch_shapes=[
            pltpu.VMEM((_E, _D, _D), jnp.float32),
            pltpu.SemaphoreType.DMA((_E,)),
        ],
        compiler_params=pltpu.CompilerParams(
            dimension_semantics=("arbitrary", "arbitrary"),
        ),
    )(h, q_W, chip_weights)
    return out.reshape(_B, _E)


# speedup vs baseline: 4.8025x; 1.1427x over previous
import functools

import jax
import jax.numpy as jnp
from jax.experimental import pallas as pl
from jax.experimental.pallas import tpu as pltpu

_B, _S, _D, _E = 2, 2048, 1024, 8
_TS = 512


def _router_kernel(h_ref, qw_ref, cw_ref, out_ref, *, n_s_tiles):
    s = pl.program_id(1)

    x = h_ref[0]
    q = jax.lax.dot_general(
        x, qw_ref[...], (((1,), (1,)), ((), ())),
        precision=jax.lax.Precision.DEFAULT,
        preferred_element_type=jnp.float32)
    qn2 = jnp.sum(q * q, axis=1, keepdims=True)

    lane = jax.lax.broadcasted_iota(jnp.int32, (1, _E), 1)
    acc = jnp.zeros((1, _E), dtype=jnp.float32)
    for e in range(_E):
        k = jax.lax.dot_general(
            x, cw_ref[e], (((1,), (1,)), ((), ())),
            precision=jax.lax.Precision.DEFAULT,
            preferred_element_type=jnp.float32)
        num = jnp.sum(q * k, axis=1, keepdims=True)
        kn2 = jnp.sum(k * k, axis=1, keepdims=True)
        denom = jnp.maximum(jnp.sqrt(qn2 * kn2), 1e-8)
        part = jnp.sum(num / denom) * (1.0 / _S)
        acc = acc + jnp.where(lane == e, part, 0.0)

    @pl.when(s == 0)
    def _init():
        out_ref[...] = jnp.zeros_like(out_ref)

    out_ref[...] += acc[None]


@jax.jit
def kernel(h, q_W, chip_weights):
    n_s_tiles = _S // _TS
    out = pl.pallas_call(
        functools.partial(_router_kernel, n_s_tiles=n_s_tiles),
        grid=(_B, n_s_tiles),
        in_specs=[
            pl.BlockSpec((1, _TS, _D), lambda b, s: (b, s, 0)),
            pl.BlockSpec((_D, _D), lambda b, s: (0, 0)),
            pl.BlockSpec((_E, _D, _D), lambda b, s: (0, 0, 0)),
        ],
        out_specs=pl.BlockSpec((1, 1, _E), lambda b, s: (b, 0, 0)),
        out_shape=jax.ShapeDtypeStruct((_B, 1, _E), jnp.float32),
        compiler_params=pltpu.CompilerParams(
            dimension_semantics=("parallel", "arbitrary"),
        ),
    )(h, q_W, chip_weights)
    return out.reshape(_B, _E)
